# Initial kernel scaffold; baseline (speedup 1.0000x reference)
import jax
import jax.numpy as jnp
from jax.experimental import pallas as pl


def kernel(qkv, embedding_weight):
    raise NotImplementedError("write your pallas kernel here")


# speedup vs baseline: 1.4503x; 1.4503x over previous
import jax
import jax.numpy as jnp
from jax.experimental import pallas as pl

K = 512
D = 16
EPS = 1e-15
_PREC = jax.lax.Precision.HIGHEST


def _dot(a, b):
    return jax.lax.dot_general(a, b, ((((1,), (0,))), ((), ())),
                               preferred_element_type=jnp.float32,
                               precision=_PREC)


def _vq_attn_kernel(q_ref, kt_ref, v_aug_ref, e_ref, et_ref, o_ref):
    e = e_ref[...]
    e_sq = jnp.sum(e * e, axis=1, keepdims=True)
    e_aug = jnp.concatenate([e * (-2.0), e_sq], axis=1)

    kt = kt_ref[...]
    ones_row = jnp.ones((1, kt.shape[1]), jnp.float32)
    kt_aug = jnp.concatenate([kt, ones_row], axis=0)
    dist_t = _dot(e_aug, kt_aug)

    min_d = jnp.min(dist_t, axis=0, keepdims=True)
    code_iota = jax.lax.broadcasted_iota(jnp.int32, dist_t.shape, 0)
    idx = jnp.min(jnp.where(dist_t == min_d, code_iota, K),
                  axis=0, keepdims=True)
    one_hot_t = (code_iota == idx).astype(jnp.float32)

    otv_aug = _dot(one_hot_t, v_aug_ref[...])

    p = jnp.exp(_dot(q_ref[...], et_ref[...]))
    num_aug = _dot(p, otv_aug)
    o_ref[...] = num_aug[:, :D] / (num_aug[:, D:D + 1] + EPS)


def kernel(qkv, embedding_weight):
    B, C, H, W = qkv.shape
    qkv = qkv.astype(jnp.float32)
    x = jnp.swapaxes(jnp.reshape(qkv, (B, -1, 3 * D, H * W)), -1, -2)
    q = x[..., 0:D].reshape(-1, D)
    kt = x[..., D:2 * D].reshape(-1, D).T
    v = x[..., 2 * D:].reshape(-1, D)
    n = q.shape[0]
    v_aug = jnp.concatenate([v, jnp.ones((n, 1), jnp.float32)], axis=1)
    e = embedding_weight.astype(jnp.float32)

    out = pl.pallas_call(
        _vq_attn_kernel,
        out_shape=jax.ShapeDtypeStruct((n, D), jnp.float32),
    )(q, kt, v_aug, e, e.T)
    return jnp.reshape(out, (B, -1, H, W))


# speedup vs baseline: 1.8217x; 1.2561x over previous
import jax
import jax.numpy as jnp
from jax.experimental import pallas as pl
from jax.experimental.pallas import tpu as pltpu

K = 512
D = 16
EPS = 1e-15
_PREC = jax.lax.Precision.HIGHEST


def _dot(a, b):
    return jax.lax.dot_general(a, b, ((((1,), (0,))), ((), ())),
                               preferred_element_type=jnp.float32,
                               precision=_PREC)


def _vq_attn_kernel(x_ref, e_ref, o_ref, q_scr, kta_scr, va_scr):
    nh = x_ref.shape[0]
    hw = x_ref.shape[2]
    for h in range(nh):
        blk = x_ref[h]
        q_scr[pl.ds(h * hw, hw), :] = jnp.transpose(blk[0:D, :])
        kta_scr[0:D, pl.ds(h * hw, hw)] = blk[D:2 * D, :]
        va_scr[pl.ds(h * hw, hw), 0:D] = jnp.transpose(blk[2 * D:3 * D, :])
    kta_scr[D:D + 1, :] = jnp.ones_like(kta_scr[D:D + 1, :])
    va_scr[:, D:D + 1] = jnp.ones_like(va_scr[:, D:D + 1])

    e = e_ref[...]
    e_sq = jnp.sum(e * e, axis=1, keepdims=True)
    e_aug = jnp.concatenate([e * (-2.0), e_sq], axis=1)

    dist_t = _dot(e_aug, kta_scr[...])

    min_d = jnp.min(dist_t, axis=0, keepdims=True)
    code_iota = jax.lax.broadcasted_iota(jnp.int32, dist_t.shape, 0)
    idx = jnp.min(jnp.where(dist_t == min_d, code_iota, K),
                  axis=0, keepdims=True)
    one_hot_t = (code_iota == idx).astype(jnp.float32)

    otv_aug = _dot(one_hot_t, va_scr[...])

    p = jnp.exp(_dot(q_scr[...], jnp.transpose(e)))
    num_aug = _dot(p, otv_aug)
    o_ref[...] = num_aug[:, :D] / (num_aug[:, D:D + 1] + EPS)


def kernel(qkv, embedding_weight):
    B, C, H, W = qkv.shape
    nh = B * C // (3 * D)
    n = nh * H * W
    x = jnp.reshape(qkv.astype(jnp.float32), (nh, 3 * D, H * W))

    out = pl.pallas_call(
        _vq_attn_kernel,
        out_shape=jax.ShapeDtypeStruct((n, D), jnp.float32),
        scratch_shapes=[
            pltpu.VMEM((n, D), jnp.float32),
            pltpu.VMEM((D + 1, n), jnp.float32),
            pltpu.VMEM((n, D + 1), jnp.float32),
        ],
    )(x, embedding_weight.astype(jnp.float32))
    return jnp.reshape(out, (B, -1, H, W))
